# baseline (speedup 1.0000x reference)
import functools

import jax
import jax.numpy as jnp
from jax import lax
from jax.experimental import pallas as pl
from jax.experimental.pallas import tpu as pltpu
from jax.experimental.pallas import tpu_sc as plsc

EMBED = 64
G = 128


@functools.partial(jax.jit, static_argnames=("B", "D", "NW", "K"))
def _gather_sc(x2, table, *, B, D, NW, K):
    NB = B // G
    nb_per_w = NB // NW
    n_chunks = nb_per_w // K
    mesh = plsc.VectorSubcoreMesh(core_axis_name="c", subcore_axis_name="s")
    info = plsc.get_sparse_core_info()
    nc = info.num_cores

    @functools.partial(
        pl.kernel,
        mesh=mesh,
        out_type=jax.ShapeDtypeStruct((B, 2 * D), jnp.float32),
        scratch_types=[
            pltpu.VMEM((K, G), jnp.int32),
            pltpu.VMEM((K, G, D), jnp.float32),
            pltpu.SemaphoreType.DMA,
        ],
        compiler_params=pltpu.CompilerParams(use_tc_tiling_on_sc=False),
    )
    def k(x_hbm, table_hbm, out_hbm, idx_v, rows_v, gsem):
        wid = lax.axis_index("s") * nc + lax.axis_index("c")
        row0 = wid * nb_per_w

        def body(i, carry):
            r = row0 + i * K
            pltpu.sync_copy(x_hbm.at[pl.ds(r, K)], idx_v)
            waits = [
                pltpu.async_copy(table_hbm.at[idx_v.at[j]], rows_v.at[j], gsem)
                for j in range(K)
            ]
            for j, w in enumerate(waits):
                w.wait()
                pltpu.sync_copy(
                    rows_v.at[j],
                    out_hbm.at[pl.ds((r + j) * G, G), pl.ds(0, D)],
                )
            return carry

        lax.fori_loop(0, n_chunks, body, 0)

    return k(x2, table)


@functools.partial(jax.jit, static_argnames=("B", "D", "R"))
def _trim_tc(out128, *, B, D, R):
    def body(i_ref, o_ref):
        o_ref[...] = i_ref[:, :D]

    return pl.pallas_call(
        body,
        grid=(B // R,),
        in_specs=[pl.BlockSpec((R, 2 * D), lambda i: (i, 0))],
        out_specs=pl.BlockSpec((R, D), lambda i: (i, 0)),
        out_shape=jax.ShapeDtypeStruct((B, D), jnp.float32),
    )(out128)


def kernel(x, table):
    B = x.shape[0] * x.shape[1]
    out128 = _gather_sc(x.reshape(B // G, G), table, B=B, D=EMBED, NW=32, K=8)
    out = _trim_tc(out128, B=B, D=EMBED, R=4096)
    return out.reshape(x.shape[0], x.shape[1], EMBED)


# speedup vs baseline: 1.2379x; 1.2379x over previous
import functools

import jax
import jax.numpy as jnp
from jax import lax
from jax.experimental import pallas as pl
from jax.experimental.pallas import tpu as pltpu
from jax.experimental.pallas import tpu_sc as plsc

EMBED = 64
G = 128


@functools.partial(jax.jit, static_argnames=("B", "D", "NW", "K"))
def _gather_sc(x, table, *, B, D, NW, K):
    NB = B // G
    nb_per_w = NB // NW
    n_chunks = nb_per_w // K
    H = x.shape[1]
    xrows_per_w = x.shape[0] // NW
    n_idx = xrows_per_w * H
    assert n_idx == nb_per_w * G
    mesh = plsc.VectorSubcoreMesh(core_axis_name="c", subcore_axis_name="s")
    info = plsc.get_sparse_core_info()
    nc = info.num_cores

    @functools.partial(
        pl.kernel,
        mesh=mesh,
        out_type=jax.ShapeDtypeStruct((B, 2 * D), jnp.float32),
        scratch_types=[
            pltpu.VMEM((n_idx,), jnp.int32),
            pltpu.VMEM((K, G, D), jnp.float32),
            pltpu.SemaphoreType.DMA,
        ],
        compiler_params=pltpu.CompilerParams(use_tc_tiling_on_sc=False),
    )
    def k(x_hbm, table_hbm, out_hbm, idx_v, rows_v, gsem):
        wid = lax.axis_index("s") * nc + lax.axis_index("c")
        row0 = wid * nb_per_w
        xrow0 = wid * xrows_per_w

        def stage(j, carry):
            pltpu.sync_copy(x_hbm.at[xrow0 + j], idx_v.at[pl.ds(j * H, H)])
            return carry

        lax.fori_loop(0, xrows_per_w, stage, 0)

        def body(i, carry):
            r = row0 + i * K
            waits = [
                pltpu.async_copy(
                    table_hbm.at[idx_v.at[pl.ds((i * K + j) * G, G)]],
                    rows_v.at[j],
                    gsem,
                )
                for j in range(K)
            ]
            for j, w in enumerate(waits):
                w.wait()
                pltpu.sync_copy(
                    rows_v.at[j],
                    out_hbm.at[pl.ds((r + j) * G, G), pl.ds(0, D)],
                )
            return carry

        lax.fori_loop(0, n_chunks, body, 0)

    return k(x, table)


def kernel(x, table):
    B = x.shape[0] * x.shape[1]
    out = _gather_sc(x, table, B=B, D=EMBED, NW=32, K=8)
    return out.reshape(x.shape[0], x.shape[1], 2 * EMBED)[:, :, :EMBED]


# speedup vs baseline: 1.3034x; 1.0529x over previous
import functools

import jax
import jax.numpy as jnp
from jax import lax
from jax.experimental import pallas as pl
from jax.experimental.pallas import tpu as pltpu
from jax.experimental.pallas import tpu_sc as plsc

EMBED = 64
G = 128
HPAD = 256


@functools.partial(jax.jit, static_argnames=("B", "H", "D", "NW", "K"))
def _gather_sc(x256, table, *, B, H, D, NW, K):
    NB = B // G
    nb_per_w = NB // NW
    n_chunks = nb_per_w // K
    xrows_per_w = x256.shape[0] // NW
    n_idx = xrows_per_w * H
    assert n_idx == nb_per_w * G
    mesh = plsc.VectorSubcoreMesh(core_axis_name="c", subcore_axis_name="s")
    info = plsc.get_sparse_core_info()
    nc = info.num_cores

    @functools.partial(
        pl.kernel,
        mesh=mesh,
        out_type=jax.ShapeDtypeStruct((B, 2 * D), jnp.float32),
        scratch_types=[
            pltpu.VMEM((n_idx,), jnp.int32),
            pltpu.VMEM((2, K, G, D), jnp.float32),
            pltpu.SemaphoreType.DMA,
            pltpu.SemaphoreType.DMA,
            pltpu.SemaphoreType.DMA,
        ],
        compiler_params=pltpu.CompilerParams(use_tc_tiling_on_sc=False),
    )
    def k(x_hbm, table_hbm, out_hbm, idx_v, rows_v, isem, gsem, ssem):
        wid = lax.axis_index("s") * nc + lax.axis_index("c")
        row0 = wid * nb_per_w
        xrow0 = wid * xrows_per_w

        def stage(j, carry):
            copies = [
                pltpu.async_copy(
                    x_hbm.at[xrow0 + j * 8 + u, pl.ds(0, H)],
                    idx_v.at[pl.ds((j * 8 + u) * H, H)],
                    isem,
                )
                for u in range(8)
            ]
            for c in copies:
                c.wait()
            return carry

        lax.fori_loop(0, xrows_per_w // 8, stage, 0)

        def fire(i, b):
            return [
                pltpu.async_copy(
                    table_hbm.at[idx_v.at[pl.ds((i * K + j) * G, G)]],
                    rows_v.at[b, j],
                    gsem,
                )
                for j in range(K)
            ]

        def store(i, b):
            return [
                pltpu.async_copy(
                    rows_v.at[b, j],
                    out_hbm.at[pl.ds((row0 + i * K + j) * G, G), pl.ds(0, D)],
                    ssem,
                )
                for j in range(K)
            ]

        for w in fire(0, 0):
            w.wait()
        prev_stores = store(0, 0)

        def body(i, carry):
            b = i % 2
            gathers = fire(i, b)
            for j in range(K):
                pltpu.make_async_copy(
                    rows_v.at[1 - b, j],
                    out_hbm.at[pl.ds((row0 + (i - 1) * K + j) * G, G), pl.ds(0, D)],
                    ssem,
                ).wait()
            for w in gathers:
                w.wait()
            store(i, b)
            return carry

        lax.fori_loop(1, n_chunks, body, 0)
        b_last = (n_chunks - 1) % 2
        for j in range(K):
            pltpu.make_async_copy(
                rows_v.at[b_last, j],
                out_hbm.at[pl.ds((row0 + (n_chunks - 1) * K + j) * G, G), pl.ds(0, D)],
                ssem,
            ).wait()

    return k(x256, table)


def kernel(x, table):
    B = x.shape[0] * x.shape[1]
    x256 = jnp.pad(x, ((0, 0), (0, HPAD - x.shape[1])))
    out = _gather_sc(x256, table, B=B, H=x.shape[1], D=EMBED, NW=32, K=5)
    return out.reshape(x.shape[0], x.shape[1], 2 * EMBED)[:, :, :EMBED]


# speedup vs baseline: 1.3071x; 1.0028x over previous
import functools

import jax
import jax.numpy as jnp
from jax import lax
from jax.experimental import pallas as pl
from jax.experimental.pallas import tpu as pltpu
from jax.experimental.pallas import tpu_sc as plsc

EMBED = 64
G = 128
HREM = 72


@functools.partial(jax.jit, static_argnames=("H", "D", "NW", "K"))
def _gather_sc(xa, xc, table, *, H, D, NW, K):
    nrows = xa.shape[0]
    rows_per_w = nrows // NW
    n_chunks = rows_per_w // K
    B = nrows * H
    mesh = plsc.VectorSubcoreMesh(core_axis_name="c", subcore_axis_name="s")
    info = plsc.get_sparse_core_info()
    nc = info.num_cores

    @functools.partial(
        pl.kernel,
        mesh=mesh,
        out_type=jax.ShapeDtypeStruct((B, 2 * D), jnp.float32),
        scratch_types=[
            pltpu.VMEM((rows_per_w, G), jnp.int32),
            pltpu.VMEM((rows_per_w, G), jnp.int32),
            pltpu.VMEM((2, K * H, D), jnp.float32),
            pltpu.SemaphoreType.DMA,
            pltpu.SemaphoreType.DMA,
        ],
        compiler_params=pltpu.CompilerParams(use_tc_tiling_on_sc=False),
    )
    def k(xa_hbm, xc_hbm, table_hbm, out_hbm, ia_v, ic_v, rows_v, gsem, ssem):
        wid = lax.axis_index("s") * nc + lax.axis_index("c")
        xrow0 = wid * rows_per_w
        out0 = wid * rows_per_w * H

        pltpu.sync_copy(xa_hbm.at[pl.ds(xrow0, rows_per_w)], ia_v)
        pltpu.sync_copy(xc_hbm.at[pl.ds(xrow0, rows_per_w)], ic_v)

        def fire(i, b):
            waits = []
            for u in range(K):
                r = i * K + u
                waits.append(
                    pltpu.async_copy(
                        table_hbm.at[ia_v.at[r]],
                        rows_v.at[b, pl.ds(u * H, G)],
                        gsem,
                    )
                )
                waits.append(
                    pltpu.async_copy(
                        table_hbm.at[ic_v.at[r, pl.ds(0, HREM)]],
                        rows_v.at[b, pl.ds(u * H + G, HREM)],
                        gsem,
                    )
                )
            return waits

        def store(i, b):
            pltpu.async_copy(
                rows_v.at[b],
                out_hbm.at[pl.ds(out0 + i * K * H, K * H), pl.ds(0, D)],
                ssem,
            )

        def drain_store(i, b):
            pltpu.make_async_copy(
                rows_v.at[b],
                out_hbm.at[pl.ds(out0 + i * K * H, K * H), pl.ds(0, D)],
                ssem,
            ).wait()

        for w in fire(0, 0):
            w.wait()
        store(0, 0)

        def body(i, carry):
            b = i % 2
            gathers = fire(i, b)
            drain_store(i - 1, 1 - b)
            for w in gathers:
                w.wait()
            store(i, b)
            return carry

        lax.fori_loop(1, n_chunks, body, 0)
        drain_store(n_chunks - 1, (n_chunks - 1) % 2)

    return k(xa, xc, table)


def kernel(x, table):
    H = x.shape[1]
    xa = x[:, :G]
    xc = jnp.pad(x[:, G:], ((0, 0), (0, G - (H - G))))
    out = _gather_sc(xa, xc, table, H=H, D=EMBED, NW=32, K=2)
    return out.reshape(x.shape[0], H, 2 * EMBED)[:, :, :EMBED]
